# baseline (speedup 1.0000x reference)
import jax
import jax.numpy as jnp
from jax import lax
from jax.experimental import pallas as pl
from jax.experimental.pallas import tpu as pltpu
from jax.experimental.pallas import tpu_sc as plsc

N = 10000
E = 320000
D_IN = 128
D_HID = 128
D_OUT = 40
D_PAD = 64

NC, NS = 2, 16
NW = NC * NS
EPW = E // NW
KA = 40
CHUNKSA = EPW // KA
ROW_STRIDE = 624
ROW_SPAN = 640

KB = 80
EPT = E // NS
CHUNKSB = EPT // KB
NBUF = 5
GB = CHUNKSB // NBUF
NQ = ROW_SPAN // KB

_mesh = plsc.VectorSubcoreMesh(core_axis_name="c", subcore_axis_name="s")


def _deg_body(dst_hbm, zero_hbm, out_a, out_b, ones_v, dst_v, zbuf, acc,
              sem_s):
    cid = lax.axis_index("c")
    sid = lax.axis_index("s")
    wid = sid * NC + cid
    off = sid * ROW_STRIDE
    for j in range(KB // 16):
        ones_v[pl.ds(j * 16, 16)] = jnp.full((16,), 1.0, jnp.float32)
    pltpu.sync_copy(dst_hbm.at[wid], dst_v)
    pltpu.sync_copy(zero_hbm, zbuf)
    for q in range(ROW_SPAN // KB):
        pltpu.sync_copy(zbuf.at[pl.ds(0, KB)], acc.at[pl.ds(off + q * KB, KB)])
    plsc.subcore_barrier()

    _NB = 5
    ones = ones_v.at[pl.ds(0, KA)]

    def scat(i, b):
        pltpu.async_copy(ones, acc.at[dst_v.at[i]], sem_s.at[b], add=True)

    for b in range(_NB):
        scat(b, b)

    def group(g, carry):
        for b in range(_NB):
            i = g * _NB + b
            pltpu.make_async_copy(ones, acc.at[dst_v.at[i]],
                                  sem_s.at[b]).wait()
            scat(i + _NB, b)
        return carry

    lax.fori_loop(0, CHUNKSA // _NB - 1, group, 0)
    for b in range(_NB):
        i = (CHUNKSA // _NB - 1) * _NB + b
        pltpu.make_async_copy(ones, acc.at[dst_v.at[i]],
                              sem_s.at[b]).wait()
    plsc.subcore_barrier()
    out = [out_a, out_b]
    for c in range(NC):
        @pl.when(cid == c)
        def _(c=c):
            for q in range(ROW_SPAN // KB):
                pltpu.sync_copy(acc.at[pl.ds(off + q * KB, KB)],
                                zbuf.at[pl.ds(0, KB)])
                pltpu.sync_copy(zbuf.at[pl.ds(0, KB)],
                                out[c].at[pl.ds(off + q * KB, KB)])


_deg_call = pl.kernel(
    _deg_body,
    out_type=[jax.ShapeDtypeStruct((N,), jnp.float32),
              jax.ShapeDtypeStruct((N,), jnp.float32)],
    mesh=_mesh,
    compiler_params=pltpu.CompilerParams(use_tc_tiling_on_sc=False),
    scratch_types=[
        pltpu.VMEM((KB,), jnp.float32),
        pltpu.VMEM((CHUNKSA, KA), jnp.int32),
        pltpu.VMEM((KB,), jnp.float32),
        pltpu.VMEM_SHARED((N,), jnp.float32),
        pltpu.SemaphoreType.DMA((5,)),
    ],
)


def _agg2_body(hp_hbm, idx_hbm, stg_hbm, zero_hbm, out_hbm,
               stg_v, ibuf, rows, acc, hp_sh, sem_g, sem_s, sem_i):
    cid = lax.axis_index("c")
    sid = lax.axis_index("s")
    off = sid * ROW_STRIDE

    pltpu.sync_copy(stg_hbm.at[cid, sid], stg_v)
    pltpu.async_copy(idx_hbm.at[sid, pl.ds(0, NBUF)], ibuf.at[0],
                     sem_i.at[0])
    pltpu.async_copy(idx_hbm.at[sid, pl.ds(NBUF, NBUF)], ibuf.at[1],
                     sem_i.at[1])

    pltpu.sync_copy(zero_hbm, rows.at[0])
    for q in range(NQ):
        pltpu.sync_copy(rows.at[0], acc.at[pl.ds(off + q * KB, KB)])

    pltpu.async_copy(hp_hbm.at[stg_v.at[0]], rows.at[0], sem_g.at[0])
    pltpu.async_copy(hp_hbm.at[stg_v.at[1]], rows.at[1], sem_g.at[1])
    for q in range(NQ):
        s = q % 2
        pltpu.make_async_copy(hp_hbm.at[stg_v.at[q]], rows.at[s],
                              sem_g.at[s]).wait()
        pltpu.async_copy(rows.at[s], hp_sh.at[pl.ds(off + q * KB, KB)],
                         sem_s.at[s])
        if q + 2 < NQ:
            pltpu.make_async_copy(rows.at[s],
                                  hp_sh.at[pl.ds(off + q * KB, KB)],
                                  sem_s.at[s]).wait()
            pltpu.async_copy(hp_hbm.at[stg_v.at[q + 2]], rows.at[s],
                             sem_g.at[s])
    for q in (NQ - 2, NQ - 1):
        pltpu.make_async_copy(rows.at[q % 2],
                              hp_sh.at[pl.ds(off + q * KB, KB)],
                              sem_s.at[q % 2]).wait()
    pltpu.make_async_copy(idx_hbm.at[sid, pl.ds(0, NBUF)], ibuf.at[0],
                          sem_i.at[0]).wait()
    plsc.subcore_barrier()

    def fire_gather(b, slot):
        pltpu.async_copy(hp_sh.at[ibuf.at[slot, b, 0]], rows.at[b],
                         sem_g.at[b])

    def wait_gather(b, slot):
        pltpu.make_async_copy(hp_sh.at[ibuf.at[slot, b, 0]], rows.at[b],
                              sem_g.at[b]).wait()

    def fire_scatter(b, slot):
        pltpu.async_copy(rows.at[b], acc.at[ibuf.at[slot, b, 1]],
                         sem_s.at[b], add=True)

    def wait_scatter(b, slot):
        pltpu.make_async_copy(rows.at[b], acc.at[ibuf.at[slot, b, 1]],
                              sem_s.at[b]).wait()

    def wait_idx(slot):
        pltpu.make_async_copy(idx_hbm.at[sid, pl.ds(0, NBUF)],
                              ibuf.at[slot], sem_i.at[slot]).wait()

    for b in range(NBUF):
        fire_gather(b, 0)

    def pair(k, carry):
        for b in range(NBUF):
            wait_gather(b, 0)
            fire_scatter(b, 0)
        wait_idx(1)
        for b in range(NBUF):
            wait_scatter(b, 0)
            fire_gather(b, 1)
        pltpu.async_copy(idx_hbm.at[sid, pl.ds((2 * k + 2) * NBUF, NBUF)],
                         ibuf.at[0], sem_i.at[0])
        for b in range(NBUF):
            wait_gather(b, 1)
            fire_scatter(b, 1)
        wait_idx(0)
        for b in range(NBUF):
            wait_scatter(b, 1)
            fire_gather(b, 0)
        pltpu.async_copy(idx_hbm.at[sid, pl.ds((2 * k + 3) * NBUF, NBUF)],
                         ibuf.at[1], sem_i.at[1])
        return carry

    lax.fori_loop(0, GB // 2 - 1, pair, 0)
    for b in range(NBUF):
        wait_gather(b, 0)
        fire_scatter(b, 0)
    wait_idx(1)
    for b in range(NBUF):
        wait_scatter(b, 0)
        fire_gather(b, 1)
    for b in range(NBUF):
        wait_gather(b, 1)
        fire_scatter(b, 1)
    for b in range(NBUF):
        wait_scatter(b, 1)

    plsc.subcore_barrier()
    pltpu.async_copy(acc.at[pl.ds(off, KB)], rows.at[0], sem_g.at[0])
    pltpu.async_copy(acc.at[pl.ds(off + KB, KB)], rows.at[1], sem_g.at[1])
    for q in range(NQ):
        s = q % 2
        pltpu.make_async_copy(acc.at[pl.ds(off + q * KB, KB)], rows.at[s],
                              sem_g.at[s]).wait()
        pltpu.async_copy(rows.at[s], out_hbm.at[stg_v.at[q]], sem_s.at[s])
        if q + 2 < NQ:
            pltpu.make_async_copy(rows.at[s], out_hbm.at[stg_v.at[q]],
                                  sem_s.at[s]).wait()
            pltpu.async_copy(acc.at[pl.ds(off + (q + 2) * KB, KB)],
                             rows.at[s], sem_g.at[s])
    for q in (NQ - 2, NQ - 1):
        pltpu.make_async_copy(rows.at[q % 2], out_hbm.at[stg_v.at[q]],
                              sem_s.at[q % 2]).wait()


def _make_agg2(dh):
    return pl.kernel(
        _agg2_body,
        out_type=jax.ShapeDtypeStruct((2 * N, dh), jnp.float32),
        mesh=_mesh,
        compiler_params=pltpu.CompilerParams(use_tc_tiling_on_sc=False),
        scratch_types=[
            pltpu.VMEM((NQ, KB), jnp.int32),
            pltpu.VMEM((2, NBUF, 2, KB), jnp.int32),
            pltpu.VMEM((NBUF, KB, dh), jnp.float32),
            pltpu.VMEM_SHARED((N, dh), jnp.float32),
            pltpu.VMEM_SHARED((N, dh), jnp.float32),
            pltpu.SemaphoreType.DMA((NBUF,)),
            pltpu.SemaphoreType.DMA((NBUF,)),
            pltpu.SemaphoreType.DMA((2,)),
        ],
    )


_agg2_64 = _make_agg2(64)
_agg2_32 = _make_agg2(32)


_R = 1000
_G = N // _R


def _tc1_body(x_ref, w_ref, da_ref, db_ref, hp_ref, dinv_ref):
    d = da_ref[...] + db_ref[...] + 1.0
    dinv = lax.rsqrt(d)
    dinv_ref[...] = dinv
    h = jnp.dot(x_ref[...], w_ref[...], preferred_element_type=jnp.float32)
    hp_ref[...] = h * dinv


def _tc1(x, w, da, db):
    return pl.pallas_call(
        _tc1_body,
        grid=(_G,),
        in_specs=[
            pl.BlockSpec((_R, D_IN), lambda i: (i, 0)),
            pl.BlockSpec((D_IN, D_HID), lambda i: (0, 0)),
            pl.BlockSpec((_R, 1), lambda i: (i, 0)),
            pl.BlockSpec((_R, 1), lambda i: (i, 0)),
        ],
        out_specs=[
            pl.BlockSpec((_R, D_HID), lambda i: (i, 0)),
            pl.BlockSpec((_R, 1), lambda i: (i, 0)),
        ],
        out_shape=[jax.ShapeDtypeStruct((N, D_HID), jnp.float32),
                   jax.ShapeDtypeStruct((N, 1), jnp.float32)],
    )(x, w, da, db)


def _tc_mid_body(agg_ref, hp_ref, b_ref, dinv_ref, w_ref, out_ref):
    dinv = dinv_ref[...]
    h = dinv * (agg_ref[...] + hp_ref[...]) + b_ref[...]
    h = jnp.maximum(h, 0.0)
    out_ref[...] = jnp.dot(h, w_ref[...],
                           preferred_element_type=jnp.float32) * dinv


def _tc_mid(agg, hp, b, dinv, w, d_in, d_out):
    return pl.pallas_call(
        _tc_mid_body,
        grid=(_G,),
        in_specs=[
            pl.BlockSpec((_R, d_in), lambda i: (i, 0)),
            pl.BlockSpec((_R, d_in), lambda i: (i, 0)),
            pl.BlockSpec((1, d_in), lambda i: (0, 0)),
            pl.BlockSpec((_R, 1), lambda i: (i, 0)),
            pl.BlockSpec((d_in, d_out), lambda i: (0, 0)),
        ],
        out_specs=pl.BlockSpec((_R, d_out), lambda i: (i, 0)),
        out_shape=jax.ShapeDtypeStruct((N, d_out), jnp.float32),
    )(agg, hp, b, dinv, w)


def _tc_fin_body(agg_ref, hp_ref, b_ref, dinv_ref, out_ref):
    h = dinv_ref[...] * (agg_ref[...] + hp_ref[...]) + b_ref[...]
    mask = lax.broadcasted_iota(jnp.int32, (_R, D_PAD), 1) < D_OUT
    hm = jnp.where(mask, h, -jnp.inf)
    m = jnp.max(hm, axis=1, keepdims=True)
    s = jnp.sum(jnp.where(mask, jnp.exp(h - m), 0.0), axis=1, keepdims=True)
    out_ref[...] = h - (jnp.log(s) + m)


def _tc_fin(agg, hp, b, dinv):
    return pl.pallas_call(
        _tc_fin_body,
        grid=(_G,),
        in_specs=[
            pl.BlockSpec((_R, D_PAD), lambda i: (i, 0)),
            pl.BlockSpec((_R, D_PAD), lambda i: (i, 0)),
            pl.BlockSpec((1, D_PAD), lambda i: (0, 0)),
            pl.BlockSpec((_R, 1), lambda i: (i, 0)),
        ],
        out_specs=pl.BlockSpec((_R, D_PAD), lambda i: (i, 0)),
        out_shape=jax.ShapeDtypeStruct((N, D_PAD), jnp.float32),
    )(agg, hp, b, dinv)


def kernel(x, edge_index, W1, b1, W2, b2, W3, b3):
    srcT = edge_index[0].reshape(NS, CHUNKSB, KB)
    dstT = edge_index[1].reshape(NS, CHUNKSB, KB)
    idxB = jnp.stack([srcT, dstT], axis=2)
    dstW = edge_index[1].reshape(NW, CHUNKSA, KA)
    base = (jnp.arange(NS, dtype=jnp.int32)[:, None] * ROW_STRIDE
            + jnp.arange(ROW_SPAN, dtype=jnp.int32)[None, :])
    stg = jnp.stack([2 * base, 2 * base + 1]).reshape(2, NS, NQ, KB)
    zero1 = jnp.zeros((KB,), jnp.float32)
    zero64 = jnp.zeros((KB, 64), jnp.float32)
    zero32 = jnp.zeros((KB, 32), jnp.float32)

    dega, degb = _deg_call(dstW, zero1)
    hp1, dinv = _tc1(x, W1, dega.reshape(N, 1), degb.reshape(N, 1))

    a1 = _agg2_64(hp1.reshape(2 * N, 64), idxB, stg, zero64).reshape(N, D_HID)
    hp2 = _tc_mid(a1, hp1, b1.reshape(1, D_HID), dinv, W2, D_HID, D_HID)

    a2 = _agg2_64(hp2.reshape(2 * N, 64), idxB, stg, zero64).reshape(N, D_HID)
    W3p = jnp.pad(W3, ((0, 0), (0, D_PAD - D_OUT)))
    hp3 = _tc_mid(a2, hp2, b2.reshape(1, D_HID), dinv, W3p, D_HID, D_PAD)

    a3 = _agg2_32(hp3.reshape(2 * N, 32), idxB, stg, zero32).reshape(N, D_PAD)
    b3p = jnp.pad(b3, (0, D_PAD - D_OUT)).reshape(1, D_PAD)
    out = _tc_fin(a3, hp3, b3p, dinv)
    return out[:, :D_OUT]


# speedup vs baseline: 1.3797x; 1.3797x over previous
import jax
import jax.numpy as jnp
from jax import lax
from jax.experimental import pallas as pl
from jax.experimental.pallas import tpu as pltpu
from jax.experimental.pallas import tpu_sc as plsc

N = 10000
E = 320000
D_IN = 128
D_HID = 128
D_OUT = 40
D_PAD = 64

NC, NS = 2, 16
NW = NC * NS
EPW = E // NW
K = 80
CHUNKS = EPW // K
KA = 40
CHUNKSA = EPW // KA
NBUF = 5
GROUPS = CHUNKSA // NBUF
ROW_STRIDE = 624
ROW_SPAN = 640

_mesh = plsc.VectorSubcoreMesh(core_axis_name="c", subcore_axis_name="s")


def _deg_body(dst_hbm, zero_hbm, out_a, out_b, ones_v, dst_v, zbuf, acc,
              sem_s):
    cid = lax.axis_index("c")
    sid = lax.axis_index("s")
    wid = sid * NC + cid
    off = sid * ROW_STRIDE
    for j in range(K // 16):
        ones_v[pl.ds(j * 16, 16)] = jnp.full((16,), 1.0, jnp.float32)
    pltpu.sync_copy(dst_hbm.at[wid], dst_v)
    pltpu.sync_copy(zero_hbm, zbuf)
    for q in range(ROW_SPAN // K):
        pltpu.sync_copy(zbuf, acc.at[pl.ds(off + q * K, K)])
    plsc.subcore_barrier()

    _NB = 5
    ones = ones_v.at[pl.ds(0, KA)]

    def scat(i, b):
        pltpu.async_copy(ones, acc.at[dst_v.at[i]], sem_s.at[b], add=True)

    for b in range(_NB):
        scat(b, b)

    def group(g, carry):
        for b in range(_NB):
            i = g * _NB + b
            pltpu.make_async_copy(ones, acc.at[dst_v.at[i]],
                                  sem_s.at[b]).wait()
            scat(i + _NB, b)
        return carry

    lax.fori_loop(0, CHUNKSA // _NB - 1, group, 0)
    for b in range(_NB):
        i = (CHUNKSA // _NB - 1) * _NB + b
        pltpu.make_async_copy(ones, acc.at[dst_v.at[i]],
                              sem_s.at[b]).wait()
    plsc.subcore_barrier()
    out = [out_a, out_b]
    for c in range(NC):
        @pl.when(cid == c)
        def _(c=c):
            for q in range(ROW_SPAN // K):
                pltpu.sync_copy(acc.at[pl.ds(off + q * K, K)], zbuf)
                pltpu.sync_copy(zbuf, out[c].at[pl.ds(off + q * K, K)])


_deg_call = pl.kernel(
    _deg_body,
    out_type=[jax.ShapeDtypeStruct((N,), jnp.float32),
              jax.ShapeDtypeStruct((N,), jnp.float32)],
    mesh=_mesh,
    compiler_params=pltpu.CompilerParams(use_tc_tiling_on_sc=False),
    scratch_types=[
        pltpu.VMEM((K,), jnp.float32),
        pltpu.VMEM((CHUNKSA, KA), jnp.int32),
        pltpu.VMEM((K,), jnp.float32),
        pltpu.VMEM_SHARED((N,), jnp.float32),
        pltpu.SemaphoreType.DMA((5,)),
    ],
)


def _agg_body(hp_hbm, src_hbm, dst_hbm, zero_hbm, out_a, out_b,
              src_v, dst_v, rows, acc, sem_g, sem_s):
    cid = lax.axis_index("c")
    sid = lax.axis_index("s")
    wid = sid * NC + cid
    off = sid * ROW_STRIDE

    pltpu.async_copy(src_hbm.at[wid], src_v, sem_g.at[0])
    pltpu.async_copy(dst_hbm.at[wid], dst_v, sem_g.at[1])

    pltpu.sync_copy(zero_hbm, rows.at[0])
    for q in range(ROW_SPAN // KA):
        pltpu.sync_copy(rows.at[0], acc.at[pl.ds(off + q * KA, KA)])
    pltpu.make_async_copy(src_hbm.at[wid], src_v, sem_g.at[0]).wait()
    pltpu.make_async_copy(dst_hbm.at[wid], dst_v, sem_g.at[1]).wait()

    def gather(i, b):
        return pltpu.async_copy(hp_hbm.at[src_v.at[i]], rows.at[b],
                                sem_g.at[b])

    def scatter(i, b):
        return pltpu.async_copy(rows.at[b], acc.at[dst_v.at[i]],
                                sem_s.at[b], add=True)

    for b in range(NBUF):
        gather(b, b)
    plsc.subcore_barrier()

    def group(g, carry):
        for b in range(NBUF):
            i = g * NBUF + b
            pltpu.make_async_copy(hp_hbm.at[src_v.at[i]], rows.at[b],
                                  sem_g.at[b]).wait()
            scatter(i, b)
        for b in range(NBUF):
            i = g * NBUF + b
            pltpu.make_async_copy(rows.at[b], acc.at[dst_v.at[i]],
                                  sem_s.at[b]).wait()
            gather(i + NBUF, b)
        return carry

    lax.fori_loop(0, GROUPS - 1, group, 0)
    for b in range(NBUF):
        i = (GROUPS - 1) * NBUF + b
        pltpu.make_async_copy(hp_hbm.at[src_v.at[i]], rows.at[b],
                              sem_g.at[b]).wait()
        scatter(i, b)
    for b in range(NBUF):
        i = (GROUPS - 1) * NBUF + b
        pltpu.make_async_copy(rows.at[b], acc.at[dst_v.at[i]],
                              sem_s.at[b]).wait()

    plsc.subcore_barrier()
    out = [out_a, out_b]
    nq = ROW_SPAN // KA
    for c in range(NC):
        @pl.when(cid == c)
        def _(c=c):
            def rd(q, s):
                pltpu.async_copy(acc.at[pl.ds(off + q * KA, KA)],
                                 rows.at[s], sem_g.at[s])

            def rd_wait(q, s):
                pltpu.make_async_copy(acc.at[pl.ds(off + q * KA, KA)],
                                      rows.at[s], sem_g.at[s]).wait()

            def wr(q, s):
                pltpu.async_copy(rows.at[s],
                                 out[c].at[pl.ds(off + q * KA, KA)],
                                 sem_s.at[s])

            def wr_wait(q, s):
                pltpu.make_async_copy(rows.at[s],
                                      out[c].at[pl.ds(off + q * KA, KA)],
                                      sem_s.at[s]).wait()

            rd(0, 0)
            rd(1, 1)
            for q in range(nq):
                s = q % 2
                rd_wait(q, s)
                wr(q, s)
                if q + 2 < nq:
                    wr_wait(q, s)
                    rd(q + 2, s)
            wr_wait(nq - 2, (nq - 2) % 2)
            wr_wait(nq - 1, (nq - 1) % 2)


def _make_agg(d):
    return pl.kernel(
        _agg_body,
        out_type=[jax.ShapeDtypeStruct((N, d), jnp.float32),
                  jax.ShapeDtypeStruct((N, d), jnp.float32)],
        mesh=_mesh,
        compiler_params=pltpu.CompilerParams(use_tc_tiling_on_sc=False),
        scratch_types=[
            pltpu.VMEM((CHUNKSA, KA), jnp.int32),
            pltpu.VMEM((CHUNKSA, KA), jnp.int32),
            pltpu.VMEM((NBUF, KA, d), jnp.float32),
            pltpu.VMEM_SHARED((N, d), jnp.float32),
            pltpu.SemaphoreType.DMA((NBUF,)),
            pltpu.SemaphoreType.DMA((NBUF,)),
        ],
    )


_agg128 = _make_agg(D_HID)
_agg64 = _make_agg(D_PAD)


_R = 1000
_G = N // _R


def _prep_body(da_ref, db_ref, dinv_ref):
    d = da_ref[...] + db_ref[...] + 1.0
    dinv_ref[...] = lax.rsqrt(d).reshape(N, 1)


def _prep(da, db):
    return pl.pallas_call(
        _prep_body,
        in_specs=[
            pl.BlockSpec((N,), lambda: (0,)),
            pl.BlockSpec((N,), lambda: (0,)),
        ],
        out_specs=pl.BlockSpec((N, 1), lambda: (0, 0)),
        out_shape=jax.ShapeDtypeStruct((N, 1), jnp.float32),
    )(da, db)


def _tc1_body(x_ref, w_ref, dinv_ref, hp_ref):
    h = jnp.dot(x_ref[...], w_ref[...], preferred_element_type=jnp.float32)
    hp_ref[...] = h * dinv_ref[...]


def _tc1(x, w, dinv):
    return pl.pallas_call(
        _tc1_body,
        grid=(_G,),
        in_specs=[
            pl.BlockSpec((_R, D_IN), lambda i: (i, 0)),
            pl.BlockSpec((D_IN, D_HID), lambda i: (0, 0)),
            pl.BlockSpec((_R, 1), lambda i: (i, 0)),
        ],
        out_specs=pl.BlockSpec((_R, D_HID), lambda i: (i, 0)),
        out_shape=jax.ShapeDtypeStruct((N, D_HID), jnp.float32),
    )(x, w, dinv)


def _tc_mid_body(aa_ref, ab_ref, hp_ref, b_ref, dinv_ref, w_ref, out_ref):
    dinv = dinv_ref[...]
    h = dinv * (aa_ref[...] + ab_ref[...] + hp_ref[...]) + b_ref[...]
    h = jnp.maximum(h, 0.0)
    out_ref[...] = jnp.dot(h, w_ref[...],
                           preferred_element_type=jnp.float32) * dinv


def _tc_mid(aa, ab, hp, b, dinv, w, d_in, d_out):
    return pl.pallas_call(
        _tc_mid_body,
        grid=(_G,),
        in_specs=[
            pl.BlockSpec((_R, d_in), lambda i: (i, 0)),
            pl.BlockSpec((_R, d_in), lambda i: (i, 0)),
            pl.BlockSpec((_R, d_in), lambda i: (i, 0)),
            pl.BlockSpec((1, d_in), lambda i: (0, 0)),
            pl.BlockSpec((_R, 1), lambda i: (i, 0)),
            pl.BlockSpec((d_in, d_out), lambda i: (0, 0)),
        ],
        out_specs=pl.BlockSpec((_R, d_out), lambda i: (i, 0)),
        out_shape=jax.ShapeDtypeStruct((N, d_out), jnp.float32),
    )(aa, ab, hp, b, dinv, w)


def _tc_fin_body(aa_ref, ab_ref, hp_ref, b_ref, dinv_ref, out_ref):
    dinv = dinv_ref[...]
    h = dinv * (aa_ref[...] + ab_ref[...] + hp_ref[...]) + b_ref[...]
    mask = lax.broadcasted_iota(jnp.int32, (_R, D_PAD), 1) < D_OUT
    hm = jnp.where(mask, h, -jnp.inf)
    m = jnp.max(hm, axis=1, keepdims=True)
    s = jnp.sum(jnp.where(mask, jnp.exp(h - m), 0.0), axis=1, keepdims=True)
    out_ref[...] = h - (jnp.log(s) + m)


def _tc_fin(aa, ab, hp, b, dinv):
    return pl.pallas_call(
        _tc_fin_body,
        grid=(_G,),
        in_specs=[
            pl.BlockSpec((_R, D_PAD), lambda i: (i, 0)),
            pl.BlockSpec((_R, D_PAD), lambda i: (i, 0)),
            pl.BlockSpec((_R, D_PAD), lambda i: (i, 0)),
            pl.BlockSpec((1, D_PAD), lambda i: (0, 0)),
            pl.BlockSpec((_R, 1), lambda i: (i, 0)),
        ],
        out_specs=pl.BlockSpec((_R, D_PAD), lambda i: (i, 0)),
        out_shape=jax.ShapeDtypeStruct((N, D_PAD), jnp.float32),
    )(aa, ab, hp, b, dinv)


def kernel(x, edge_index, W1, b1, W2, b2, W3, b3):
    src = edge_index[0].reshape(NW, CHUNKSA, KA)
    dst = edge_index[1].reshape(NW, CHUNKSA, KA)
    zero1 = jnp.zeros((K,), jnp.float32)
    zero128 = jnp.zeros((KA, D_HID), jnp.float32)
    zero64 = jnp.zeros((KA, D_PAD), jnp.float32)

    dega, degb = _deg_call(dst, zero1)
    dinv = _prep(dega, degb)
    hp1 = _tc1(x, W1, dinv)

    a1a, a1b = _agg128(hp1, src, dst, zero128)
    hp2 = _tc_mid(a1a, a1b, hp1, b1.reshape(1, D_HID), dinv, W2,
                  D_HID, D_HID)

    a2a, a2b = _agg128(hp2, src, dst, zero128)
    W3p = jnp.pad(W3, ((0, 0), (0, D_PAD - D_OUT)))
    hp3 = _tc_mid(a2a, a2b, hp2, b2.reshape(1, D_HID), dinv, W3p,
                  D_HID, D_PAD)

    a3a, a3b = _agg64(hp3, src, dst, zero64)
    b3p = jnp.pad(b3, (0, D_PAD - D_OUT)).reshape(1, D_PAD)
    out = _tc_fin(a3a, a3b, hp3, b3p, dinv)
    return out[:, :D_OUT]
